# baseline (speedup 1.0000x reference)
import functools

import jax
import jax.numpy as jnp
from jax import lax
from jax.experimental import pallas as pl
from jax.experimental.pallas import tpu as pltpu
from jax.experimental.pallas import tpu_sc as plsc

N = 10000
E = 320000
D = 128
DE = 16
H = 128
OUT = 2

NC = 2
NS = 16
NW = NC * NS
CH = 128
KA = 104
KB = 56
CF = 0
NCHP = 2816
EP = NCHP * CH
AR = 10112
ZR = AR // NS
DR = 80
KD = NCHP // NW

_BASE_B = NS * KA
_PH0 = (56, 32)
_PH1 = (48, 24)
_T14 = E // CH - (_BASE_B + 14 * KB)
assert _T14 == 52 and (E % CH) == 0


def _deg_body(dst2, zeros, ident, outdeg, accdeg, didx, degv, identv):
    c = lax.axis_index("c")
    s = lax.axis_index("s")
    t = s * NC + c
    pltpu.sync_copy(zeros.at[pl.ds(0, DR)], degv)

    @pl.when(s == 0)
    def _():
        pltpu.sync_copy(zeros.at[pl.ds(0, DR)], accdeg)

    pltpu.sync_copy(ident, identv)
    pltpu.sync_copy(dst2.at[pl.ds(t * KD, KD)], didx)
    plsc.subcore_barrier()

    def body(k, carry):
        for g in range(CH // 16):
            d16 = didx[k, pl.ds(g * 16, 16)]
            cnt, last = plsc.scan_count(d16)
            plsc.addupdate_scatter(
                degv,
                [lax.shift_right_logical(d16, 7), lax.bitwise_and(d16, 127)],
                cnt.astype(jnp.float32), mask=last)
        return carry

    lax.fori_loop(0, KD, body, 0)
    pltpu.sync_copy(degv, accdeg.at[identv], add=True)
    plsc.subcore_barrier()

    @pl.when(s == 0)
    def _():
        pltpu.sync_copy(accdeg, outdeg.at[c])


def _seg_pipe(isfast, base, table, src2, dst2, acc, sidx, didx,
              rows0, rows1, sem0, sem1):
    off = base
    for ph, (fa, sl) in enumerate((_PH0, _PH1)):
        nh = fa if fa == sl else jnp.where(isfast, fa, sl)
        st = pl.multiple_of(off, 8)
        pltpu.sync_copy(src2.at[pl.ds(st, fa)], sidx.at[pl.ds(0, fa)])
        pltpu.sync_copy(dst2.at[pl.ds(st, fa)], didx.at[pl.ds(0, fa)])
        off = off + nh
        if ph == 0:
            plsc.subcore_barrier()
        def body(k, carry):
            pltpu.sync_copy(table.at[sidx.at[k]], rows0)
            pltpu.sync_copy(rows0, acc.at[didx.at[k]], add=True)
            return carry

        lax.fori_loop(0, nh, body, 0)


def _seg_sum_body(table, src2, dst2, zeros, out,
                  acc, sidx, didx, rows0, rows1, sem0, sem1):
    c = lax.axis_index("c")
    s = lax.axis_index("s")
    isfast = c == CF
    base = jnp.where(isfast, s * KA, _BASE_B + s * KB)
    pltpu.sync_copy(zeros, acc.at[pl.ds(s * ZR, ZR)])
    _seg_pipe(isfast, base, table, src2, dst2, acc, sidx, didx,
              rows0, rows1, sem0, sem1)
    plsc.subcore_barrier()
    pltpu.sync_copy(acc.at[pl.ds(s * ZR, ZR)], out.at[c, pl.ds(s * ZR, ZR)])


def _cmb_loop(nk, pv, sidx, didx, o0v, o1v):
    lanes = lax.iota(jnp.int32, 16)

    def body(k, carry):
        for g in range(CH // 16):
            s16 = sidx[k, pl.ds(g * 16, 16)] * 4
            d16 = didx[k, pl.ds(g * 16, 16)] * 4
            row = k * CH + g * 16 + lanes
            a0 = plsc.load_gather(pv, [s16])
            a1 = plsc.load_gather(pv, [s16 + 1])
            b0 = plsc.load_gather(pv, [d16 + 2])
            b1 = plsc.load_gather(pv, [d16 + 3])
            plsc.store_scatter(o0v, [row], a0 + b0)
            plsc.store_scatter(o1v, [row], a1 + b1)
        return carry

    lax.fori_loop(0, nk, body, 0)


def _combine_body(ptab, src2, dst2, g0, g1, pv, sidx, didx, o0v, o1v):
    c = lax.axis_index("c")
    s = lax.axis_index("s")
    isfast = c == CF
    base = pl.multiple_of(jnp.where(isfast, s * KA, _BASE_B + s * KB), 8)
    nk = KA if KA == KB else jnp.where(isfast, KA, KB)
    pltpu.sync_copy(ptab, pv)
    pltpu.sync_copy(src2.at[pl.ds(base, KA)], sidx)
    pltpu.sync_copy(dst2.at[pl.ds(base, KA)], didx)
    _cmb_loop(nk, pv, sidx, didx, o0v, o1v)

    @pl.when(isfast)
    def _():
        st = pl.multiple_of(base * CH, 8)
        pltpu.sync_copy(o0v, g0.at[pl.ds(st, KA * CH)])
        pltpu.sync_copy(o1v, g1.at[pl.ds(st, KA * CH)])

    @pl.when(jnp.logical_not(isfast) & (s < 14))
    def _():
        st = pl.multiple_of(base * CH, 8)
        pltpu.sync_copy(o0v.at[pl.ds(0, KB * CH)], g0.at[pl.ds(st, KB * CH)])
        pltpu.sync_copy(o1v.at[pl.ds(0, KB * CH)], g1.at[pl.ds(st, KB * CH)])

    @pl.when(jnp.logical_not(isfast) & (s == 14))
    def _():
        st = pl.multiple_of(base * CH, 8)
        pltpu.sync_copy(o0v.at[pl.ds(0, _T14 * CH)],
                        g0.at[pl.ds(st, _T14 * CH)])
        pltpu.sync_copy(o1v.at[pl.ds(0, _T14 * CH)],
                        g1.at[pl.ds(st, _T14 * CH)])


@functools.lru_cache(maxsize=None)
def _sc_kernels():
    mesh = plsc.VectorSubcoreMesh(core_axis_name="c", subcore_axis_name="s",
                                  num_cores=NC, num_subcores=NS)
    params = pltpu.CompilerParams(needs_layout_passes=False)
    deg_count = pl.kernel(
        _deg_body,
        out_type=jax.ShapeDtypeStruct((NC, DR, 128), jnp.float32),
        mesh=mesh,
        scratch_types=[
            pltpu.VMEM_SHARED((DR, 128), jnp.float32),
            pltpu.VMEM((KD, CH), jnp.int32),
            pltpu.VMEM((DR, 128), jnp.float32),
            pltpu.VMEM((DR,), jnp.int32),
        ],
        compiler_params=params,
    )
    seg_sum = pl.kernel(
        _seg_sum_body,
        out_type=jax.ShapeDtypeStruct((NC, AR, H), jnp.float32),
        mesh=mesh,
        scratch_types=[
            pltpu.VMEM_SHARED((AR, H), jnp.float32),
            pltpu.VMEM((56, CH), jnp.int32),
            pltpu.VMEM((56, CH), jnp.int32),
            pltpu.VMEM((CH, H), jnp.float32),
            pltpu.VMEM((CH, H), jnp.float32),
            pltpu.SemaphoreType.DMA,
            pltpu.SemaphoreType.DMA,
        ],
        compiler_params=params,
    )
    combine = pl.kernel(
        _combine_body,
        out_type=(jax.ShapeDtypeStruct((E,), jnp.float32),
                  jax.ShapeDtypeStruct((E,), jnp.float32)),
        mesh=mesh,
        scratch_types=[
            pltpu.VMEM((AR * 4,), jnp.float32),
            pltpu.VMEM((KA, CH), jnp.int32),
            pltpu.VMEM((KA, CH), jnp.int32),
            pltpu.VMEM((KA * CH,), jnp.float32),
            pltpu.VMEM((KA * CH,), jnp.float32),
        ],
        compiler_params=params,
    )
    return deg_count, seg_sum, combine


_R = 1000


def _sage1_body(p0, p1, xe, d0, d1, w, b, o):
    se = p0[...] + p1[...] + xe[...]
    rec = 1.0 / (d0[...] + d1[...] + 1.0)
    hn = se * rec
    o[...] = jnp.maximum(
        jnp.dot(hn, w[...], preferred_element_type=jnp.float32) + b[...], 0.0)


def _sage2_body(p0, p1, h1, d0, d1, w, b, wcab, o):
    se = p0[...] + p1[...] + h1[...]
    rec = 1.0 / (d0[...] + d1[...] + 1.0)
    hn = se * rec
    ne = jnp.dot(hn, w[...], preferred_element_type=jnp.float32) + b[...]
    hnode = jnp.maximum(ne, 0.0)
    o[...] = jnp.dot(hnode, wcab[...], preferred_element_type=jnp.float32)


def _final_body(ef, g0, g1, we, bet, wccx, bcp, o):
    i = pl.program_id(0)
    het = jnp.maximum(
        lax.dot_general(we[...], ef[...], (((0,), (1,)), ((), ())),
                        preferred_element_type=jnp.float32) + bet[...], 0.0)
    aug = jnp.concatenate([het, g0[pl.ds(i * _ER, _ER)].reshape(1, _ER),
                           g1[pl.ds(i * _ER, _ER)].reshape(1, _ER)], axis=0)
    res = lax.dot_general(aug, wccx[...], (((0,), (0,)), ((), ())),
                          preferred_element_type=jnp.float32)
    o[...] = res + bcp[...]


def _node_spec():
    return pl.BlockSpec((_R, H), lambda i: (i, 0))


def _deg_spec():
    return pl.BlockSpec((_R, 1), lambda i: (i, 0))


def _full(shape):
    return pl.BlockSpec(shape, lambda i: tuple(0 for _ in shape))


_sage1 = pl.pallas_call(
    _sage1_body,
    grid=(N // _R,),
    in_specs=[_node_spec(), _node_spec(), _node_spec(),
              _deg_spec(), _deg_spec(), _full((H, H)), _full((1, H))],
    out_specs=_node_spec(),
    out_shape=jax.ShapeDtypeStruct((N, H), jnp.float32),
)

_sage2 = pl.pallas_call(
    _sage2_body,
    grid=(N // _R,),
    in_specs=[_node_spec(), _node_spec(), _node_spec(),
              _deg_spec(), _deg_spec(),
              _full((H, H)), _full((1, H)), _full((H, 4))],
    out_specs=pl.BlockSpec((_R, 4), lambda i: (i, 0)),
    out_shape=jax.ShapeDtypeStruct((AR, 4), jnp.float32),
)

_ER = 3200

_final = pl.pallas_call(
    _final_body,
    grid=(E // _ER,),
    in_specs=[pl.BlockSpec((_ER, DE), lambda i: (i, 0)),
              _full((E,)), _full((E,)),
              _full((DE, H)), _full((H, 1)), _full((H + 2, OUT)),
              _full((1, OUT))],
    out_specs=pl.BlockSpec((_ER, OUT), lambda i: (i, 0)),
    out_shape=jax.ShapeDtypeStruct((E, OUT), jnp.float32),
)


def kernel(node_feats, edge_index, edge_feats, W1, b1, W2, b2, We, be, Wc, bc):
    f32 = jnp.float32
    src = edge_index[0].astype(jnp.int32)
    dst = edge_index[1].astype(jnp.int32)
    pad = EP - E
    fill = jnp.full((pad,), N, jnp.int32)
    src2 = jnp.concatenate([src, fill]).reshape(NCHP, CH)
    dst2 = jnp.concatenate([dst, fill]).reshape(NCHP, CH)

    x = node_feats.astype(f32)
    x_pad = jnp.pad(x, ((0, AR - N), (0, 0)))
    zeros_blk = jnp.zeros((ZR, H), f32)
    ident = jnp.arange(DR, dtype=jnp.int32)

    deg_count, seg_sum, combine = _sc_kernels()
    degp = deg_count(dst2, zeros_blk, ident)
    deg0 = degp[0].reshape(DR * 128)[:N].reshape(N, 1)
    deg1 = degp[1].reshape(DR * 128)[:N].reshape(N, 1)
    parts1 = seg_sum(x_pad, src2, dst2, zeros_blk)
    h1 = _sage1(parts1[0], parts1[1], x, deg0, deg1, W1, b1.reshape(1, H))
    h1_pad = jnp.pad(h1, ((0, AR - N), (0, 0)))
    parts2 = seg_sum(h1_pad, src2, dst2, zeros_blk)

    wcab = jnp.concatenate([Wc[0:H], Wc[H:2 * H]], axis=1)
    ptab = _sage2(parts2[0], parts2[1], h1, deg0, deg1, W2,
                  b2.reshape(1, H), wcab)

    g0, g1 = combine(ptab.reshape(AR * 4), src2, dst2)
    wccx = jnp.concatenate([Wc[2 * H:3 * H],
                            jnp.eye(OUT, dtype=f32)], axis=0)
    return _final(edge_feats.astype(f32), g0, g1, We, be.reshape(H, 1),
                  wccx, bc.reshape(1, OUT))


# speedup vs baseline: 1.1954x; 1.1954x over previous
import functools

import jax
import jax.numpy as jnp
from jax import lax
from jax.experimental import pallas as pl
from jax.experimental.pallas import tpu as pltpu
from jax.experimental.pallas import tpu_sc as plsc

N = 10000
E = 320000
D = 128
DE = 16
H = 128
OUT = 2

NC = 2
NS = 16
NW = NC * NS
CH = 128
EPT = 10112
K = EPT // CH
EP = NW * EPT
AR = 10112
ZR = AR // NS
DR = 80

ETAIL = E - (NW - 1) * EPT


def _seg_sum_body(table, src3, dst3, zeros, ident, out, outdeg,
                  acc, accdeg, sidx, didx, rows, degv, identv):
    c = lax.axis_index("c")
    s = lax.axis_index("s")
    t = s * NC + c
    pltpu.sync_copy(zeros, acc.at[pl.ds(s * ZR, ZR)])
    pltpu.sync_copy(zeros.at[pl.ds(0, DR)], degv)

    @pl.when(s == 0)
    def _():
        pltpu.sync_copy(zeros.at[pl.ds(0, DR)], accdeg)

    pltpu.sync_copy(src3.at[t], sidx)
    pltpu.sync_copy(dst3.at[t], didx)
    pltpu.sync_copy(ident, identv)
    plsc.subcore_barrier()

    def body(k, carry):
        for g in range(CH // 16):
            d16 = didx[k, pl.ds(g * 16, 16)]
            cnt, last = plsc.scan_count(d16)
            plsc.addupdate_scatter(
                degv,
                [lax.shift_right_logical(d16, 7), lax.bitwise_and(d16, 127)],
                cnt.astype(jnp.float32), mask=last)
        pltpu.sync_copy(table.at[sidx.at[k]], rows)
        pltpu.sync_copy(rows, acc.at[didx.at[k]], add=True)
        return carry

    lax.fori_loop(0, K, body, 0)
    pltpu.sync_copy(degv, accdeg.at[identv], add=True)
    plsc.subcore_barrier()
    pltpu.sync_copy(acc.at[pl.ds(s * ZR, ZR)], out.at[c, pl.ds(s * ZR, ZR)])

    @pl.when(s == 0)
    def _():
        pltpu.sync_copy(accdeg, outdeg.at[c])


def _cmb_loop(nk, pv, sidx, didx, o0v, o1v):
    lanes = lax.iota(jnp.int32, 16)

    def body(k, carry):
        for g in range(CH // 16):
            s16 = sidx[k, pl.ds(g * 16, 16)] * 4
            d16 = didx[k, pl.ds(g * 16, 16)] * 4
            row = k * CH + g * 16 + lanes
            a0 = plsc.load_gather(pv, [s16])
            a1 = plsc.load_gather(pv, [s16 + 1])
            b0 = plsc.load_gather(pv, [d16 + 2])
            b1 = plsc.load_gather(pv, [d16 + 3])
            plsc.store_scatter(o0v, [row], a0 + b0)
            plsc.store_scatter(o1v, [row], a1 + b1)
        return carry

    lax.fori_loop(0, nk, body, 0)


def _combine_body(ptab, src3, dst3, g0, g1, pv, sidx, didx, o0v, o1v):
    c = lax.axis_index("c")
    s = lax.axis_index("s")
    t = s * NC + c
    pltpu.sync_copy(ptab, pv)
    pltpu.sync_copy(src3.at[t], sidx)
    pltpu.sync_copy(dst3.at[t], didx)
    _cmb_loop(K, pv, sidx, didx, o0v, o1v)

    @pl.when(t < NW - 1)
    def _():
        pltpu.sync_copy(o0v, g0.at[pl.ds(t * EPT, EPT)])
        pltpu.sync_copy(o1v, g1.at[pl.ds(t * EPT, EPT)])

    @pl.when(t == NW - 1)
    def _():
        pltpu.sync_copy(o0v.at[pl.ds(0, ETAIL)], g0.at[pl.ds(t * EPT, ETAIL)])
        pltpu.sync_copy(o1v.at[pl.ds(0, ETAIL)], g1.at[pl.ds(t * EPT, ETAIL)])


@functools.lru_cache(maxsize=None)
def _sc_kernels():
    mesh = plsc.VectorSubcoreMesh(core_axis_name="c", subcore_axis_name="s",
                                  num_cores=NC, num_subcores=NS)
    params = pltpu.CompilerParams(needs_layout_passes=False)
    seg_sum = pl.kernel(
        _seg_sum_body,
        out_type=(jax.ShapeDtypeStruct((NC, AR, H), jnp.float32),
                  jax.ShapeDtypeStruct((NC, DR, 128), jnp.float32)),
        mesh=mesh,
        scratch_types=[
            pltpu.VMEM_SHARED((AR, H), jnp.float32),
            pltpu.VMEM_SHARED((DR, 128), jnp.float32),
            pltpu.VMEM((K, CH), jnp.int32),
            pltpu.VMEM((K, CH), jnp.int32),
            pltpu.VMEM((CH, H), jnp.float32),
            pltpu.VMEM((DR, 128), jnp.float32),
            pltpu.VMEM((DR,), jnp.int32),
        ],
        compiler_params=params,
    )
    combine = pl.kernel(
        _combine_body,
        out_type=(jax.ShapeDtypeStruct((E,), jnp.float32),
                  jax.ShapeDtypeStruct((E,), jnp.float32)),
        mesh=mesh,
        scratch_types=[
            pltpu.VMEM((AR * 4,), jnp.float32),
            pltpu.VMEM((K, CH), jnp.int32),
            pltpu.VMEM((K, CH), jnp.int32),
            pltpu.VMEM((EPT,), jnp.float32),
            pltpu.VMEM((EPT,), jnp.float32),
        ],
        compiler_params=params,
    )
    return seg_sum, combine


_R = 1000


def _sage1_body(p0, p1, xe, d0, d1, w, b, o):
    se = p0[...] + p1[...] + xe[...]
    rec = 1.0 / (d0[...] + d1[...] + 1.0)
    hn = se * rec
    o[...] = jnp.maximum(
        jnp.dot(hn, w[...], preferred_element_type=jnp.float32) + b[...], 0.0)


def _sage2_body(p0, p1, h1, d0, d1, w, b, wcab, o):
    se = p0[...] + p1[...] + h1[...]
    rec = 1.0 / (d0[...] + d1[...] + 1.0)
    hn = se * rec
    ne = jnp.dot(hn, w[...], preferred_element_type=jnp.float32) + b[...]
    hnode = jnp.maximum(ne, 0.0)
    o[...] = jnp.dot(hnode, wcab[...], preferred_element_type=jnp.float32)


def _final_body(ef, g0, g1, we, bet, wccx, bcp, o):
    i = pl.program_id(0)
    het = jnp.maximum(
        lax.dot_general(we[...], ef[...], (((0,), (1,)), ((), ())),
                        preferred_element_type=jnp.float32) + bet[...], 0.0)
    aug = jnp.concatenate([het, g0[pl.ds(i * _ER, _ER)].reshape(1, _ER),
                           g1[pl.ds(i * _ER, _ER)].reshape(1, _ER)], axis=0)
    res = lax.dot_general(aug, wccx[...], (((0,), (0,)), ((), ())),
                          preferred_element_type=jnp.float32)
    o[...] = res + bcp[...]


def _node_spec():
    return pl.BlockSpec((_R, H), lambda i: (i, 0))


def _deg_spec():
    return pl.BlockSpec((_R, 1), lambda i: (i, 0))


def _full(shape):
    return pl.BlockSpec(shape, lambda i: tuple(0 for _ in shape))


_sage1 = pl.pallas_call(
    _sage1_body,
    grid=(N // _R,),
    in_specs=[_node_spec(), _node_spec(), _node_spec(),
              _deg_spec(), _deg_spec(), _full((H, H)), _full((1, H))],
    out_specs=_node_spec(),
    out_shape=jax.ShapeDtypeStruct((N, H), jnp.float32),
)

_sage2 = pl.pallas_call(
    _sage2_body,
    grid=(N // _R,),
    in_specs=[_node_spec(), _node_spec(), _node_spec(),
              _deg_spec(), _deg_spec(),
              _full((H, H)), _full((1, H)), _full((H, 4))],
    out_specs=pl.BlockSpec((_R, 4), lambda i: (i, 0)),
    out_shape=jax.ShapeDtypeStruct((AR, 4), jnp.float32),
)

_ER = 3200

_final = pl.pallas_call(
    _final_body,
    grid=(E // _ER,),
    in_specs=[pl.BlockSpec((_ER, DE), lambda i: (i, 0)),
              _full((E,)), _full((E,)),
              _full((DE, H)), _full((H, 1)), _full((H + 2, OUT)),
              _full((1, OUT))],
    out_specs=pl.BlockSpec((_ER, OUT), lambda i: (i, 0)),
    out_shape=jax.ShapeDtypeStruct((E, OUT), jnp.float32),
)


def kernel(node_feats, edge_index, edge_feats, W1, b1, W2, b2, We, be, Wc, bc):
    f32 = jnp.float32
    src = edge_index[0].astype(jnp.int32)
    dst = edge_index[1].astype(jnp.int32)
    pad = EP - E
    fill = jnp.full((pad,), N, jnp.int32)
    src3 = jnp.concatenate([src, fill]).reshape(NW, K, CH)
    dst3 = jnp.concatenate([dst, fill]).reshape(NW, K, CH)

    x = node_feats.astype(f32)
    x_pad = jnp.pad(x, ((0, AR - N), (0, 0)))
    zeros_blk = jnp.zeros((ZR, H), f32)
    ident = jnp.arange(DR, dtype=jnp.int32)

    seg_sum, combine = _sc_kernels()
    parts1, degp = seg_sum(x_pad, src3, dst3, zeros_blk, ident)
    deg0 = degp[0].reshape(DR * 128)[:N].reshape(N, 1)
    deg1 = degp[1].reshape(DR * 128)[:N].reshape(N, 1)
    h1 = _sage1(parts1[0], parts1[1], x, deg0, deg1, W1, b1.reshape(1, H))
    h1_pad = jnp.pad(h1, ((0, AR - N), (0, 0)))
    parts2, _ = seg_sum(h1_pad, src3, dst3, zeros_blk, ident)

    wcab = jnp.concatenate([Wc[0:H], Wc[H:2 * H]], axis=1)
    ptab = _sage2(parts2[0], parts2[1], h1, deg0, deg1, W2,
                  b2.reshape(1, H), wcab)

    g0, g1 = combine(ptab.reshape(AR * 4), src3, dst3)
    wccx = jnp.concatenate([Wc[2 * H:3 * H],
                            jnp.eye(OUT, dtype=f32)], axis=0)
    return _final(edge_feats.astype(f32), g0, g1, We, be.reshape(H, 1),
                  wccx, bc.reshape(1, OUT))
